# baseline (speedup 1.0000x reference)
import functools

import jax
import jax.numpy as jnp
from jax import lax
from jax.experimental import pallas as pl
from jax.experimental.pallas import tpu as pltpu
from jax.experimental.pallas import tpu_sc as plsc

_LOG1P_C = (
    -2.31785471e-03, 1.53418978e-02, -4.76654862e-02, 9.54856631e-02,
    -1.45864737e-01, 1.93751659e-01, -2.48784242e-01, 3.33191908e-01,
    -4.99991423e-01, 9.99999795e-01, 8.18445333e-10,
)


def _log1p_poly(w):
    acc = jnp.full_like(w, _LOG1P_C[0])
    for c in _LOG1P_C[1:]:
        acc = acc * w + jnp.float32(c)
    return acc


def _sc_conf(x0, x1, y):
    B, N = y.shape
    info = plsc.get_sparse_core_info()
    nw = info.num_cores * info.num_subcores
    rpw = B // nw
    assert B % nw == 0 and N % 16 == 0
    nchunks = N // 16
    mesh = plsc.VectorSubcoreMesh(core_axis_name="c", subcore_axis_name="s")

    @functools.partial(
        pl.kernel,
        mesh=mesh,
        compiler_params=pltpu.CompilerParams(needs_layout_passes=False),
        out_type=jax.ShapeDtypeStruct((nw, 16), jnp.float32),
        scratch_types=[
            pltpu.VMEM((N,), jnp.float32),
            pltpu.VMEM((N,), jnp.float32),
            pltpu.VMEM((N,), jnp.int32),
            pltpu.VMEM((N,), jnp.float32),
            pltpu.VMEM((16,), jnp.float32),
        ],
    )
    def body(x0h, x1h, yh, outh, vx0, vx1, vy, vmine, vout):
        wid = lax.axis_index("s") * info.num_cores + lax.axis_index("c")
        lane = lax.broadcasted_iota(jnp.int32, (16,), 0)
        res = jnp.zeros((16,), jnp.float32)
        zf = jnp.zeros((16,), jnp.float32)
        for rl in range(rpw):
            r = wid * rpw + rl
            pltpu.sync_copy(x0h.at[r], vx0)
            pltpu.sync_copy(x1h.at[r], vx1)
            pltpu.sync_copy(yh.at[r], vy)

            def main_it(i, carry):
                npv, pcev, msv, zcv = carry
                sl = pl.ds(i * 16, 16)
                a0 = vx0[sl]
                a1 = vx1[sl]
                yy = vy[sl]
                t = a0 - a1
                pos = yy > 0
                ur = jnp.where(pos, jnp.maximum(t, 0.0), jnp.maximum(-t, 0.0))
                w = jnp.exp(-jnp.abs(t))
                ce = ur + _log1p_poly(w)
                mine = jnp.where(pos, jnp.float32(0.0), ce)
                vmine[sl] = mine
                one = jnp.float32(1.0)
                zero = jnp.float32(0.0)
                npv = npv + jnp.where(pos, one, zero)
                pcev = pcev + jnp.where(pos, ce, zero)
                msv = msv + mine
                zcv = zcv + jnp.where(mine == 0.0, one, zero)
                return npv, pcev, msv, zcv

            npv, pcev, msv, zcv = lax.fori_loop(
                0, nchunks, main_it, (zf, zf, zf, zf))
            np_s = jnp.sum(npv)
            pce_s = jnp.sum(pcev)
            ms_s = jnp.sum(msv)
            zc_s = jnp.sum(zcv)
            k = jnp.minimum(jnp.float32(3.0) * np_s, jnp.float32(N - 1))
            j = jnp.float32(N) - k

            def tail_zero():
                return jnp.float32(0.0)

            def tail_bisect():
                def bis(_, lohi):
                    lo, hi = lohi
                    mid = lo + (hi - lo) // 2

                    def cnt(i, acc):
                        kv = plsc.bitcast(vmine[pl.ds(i * 16, 16)], jnp.int32)
                        return acc + jnp.where(
                            kv < mid, jnp.float32(1.0), jnp.float32(0.0))

                    c = jnp.sum(lax.fori_loop(0, nchunks, cnt, zf))
                    take_hi = c < j
                    return (jnp.where(take_hi, mid, lo),
                            jnp.where(take_hi, hi, mid))

                lo, _ = lax.fori_loop(
                    0, 31, bis, (jnp.int32(0), jnp.int32(0x7F800001)))

                def fin(i, acc):
                    sb, cb = acc
                    mv = vmine[pl.ds(i * 16, 16)]
                    kv = plsc.bitcast(mv, jnp.int32)
                    ltm = kv < lo
                    sb = sb + jnp.where(ltm, mv, jnp.float32(0.0))
                    cb = cb + jnp.where(ltm, jnp.float32(1.0), jnp.float32(0.0))
                    return sb, cb

                sbv, cbv = lax.fori_loop(0, nchunks, fin, (zf, zf))
                vj = lax.bitcast_convert_type(lo, jnp.float32)
                return jnp.sum(sbv) + (j - jnp.sum(cbv)) * vj

            ss = lax.cond(j <= zc_s, tail_zero, tail_bisect)
            contrib = pce_s + (ms_s - ss)
            res = jnp.where(lane == rl, contrib, res)
            res = jnp.where(lane == rpw + rl, np_s, res)
        vout[...] = res
        pltpu.sync_copy(vout, outh.at[wid])

    return body(x0, x1, y)


def _sl1(x):
    ax = jnp.abs(x)
    return jnp.where(ax < 1.0, 0.5 * x * x, ax - 0.5)


def _tc_masked_sums(lpF, ltF, mpF, mtF, y):
    B, N = y.shape
    gb = B // 8
    grid = (gb,)

    def fold_mat(flat_ch, d):
        a_ch = flat_ch // d
        i0 = lax.broadcasted_iota(jnp.int32, (flat_ch, a_ch), 0)
        i1 = lax.broadcasted_iota(jnp.int32, (flat_ch, a_ch), 1)
        return (i0 // d == i1).astype(jnp.float32)

    def masked_fold(e, posm, d, flat_ch):
        a_ch = flat_ch // d
        nfull = (d * N) // flat_ch
        rem_flat = d * N - nfull * flat_ch
        R = fold_mat(flat_ch, d)
        acc = jnp.zeros((8, a_ch), jnp.float32)
        for c in range(nfull):
            ch = lax.slice(e, (0, c * flat_ch), (8, (c + 1) * flat_ch))
            pm = lax.slice(posm, (0, c * a_ch), (8, (c + 1) * a_ch))
            acc = acc + jnp.dot(
                ch, R, preferred_element_type=jnp.float32) * pm
        total = jnp.sum(acc)
        if rem_flat:
            Rr = fold_mat(rem_flat, d)
            ch = lax.slice(e, (0, nfull * flat_ch), (8, d * N))
            pm = lax.slice(posm, (0, nfull * a_ch), (8, N))
            total = total + jnp.sum(
                jnp.dot(ch, Rr, preferred_element_type=jnp.float32) * pm)
        return total

    def body(lp_ref, lt_ref, mp_ref, mt_ref, y_ref, ol_ref, om_ref):
        i = pl.program_id(0)
        posm = (y_ref[...] > 0).astype(jnp.float32)
        el = _sl1(lp_ref[...] - lt_ref[...])
        ol_ref[0, i] = masked_fold(el, posm, 4, 2048)
        em = _sl1(mp_ref[...] - mt_ref[...])
        om_ref[0, i] = masked_fold(em, posm, 10, 2560)

    return pl.pallas_call(
        body,
        grid=grid,
        in_specs=[
            pl.BlockSpec((8, 4 * N), lambda i: (i, 0)),
            pl.BlockSpec((8, 4 * N), lambda i: (i, 0)),
            pl.BlockSpec((8, 10 * N), lambda i: (i, 0)),
            pl.BlockSpec((8, 10 * N), lambda i: (i, 0)),
            pl.BlockSpec((8, N), lambda i: (i, 0)),
        ],
        out_specs=[
            pl.BlockSpec((1, gb), lambda i: (0, 0), memory_space=pltpu.SMEM),
            pl.BlockSpec((1, gb), lambda i: (0, 0), memory_space=pltpu.SMEM),
        ],
        out_shape=[
            jax.ShapeDtypeStruct((1, gb), jnp.float32),
            jax.ShapeDtypeStruct((1, gb), jnp.float32),
        ],
        compiler_params=pltpu.CompilerParams(
            dimension_semantics=("arbitrary",)),
    )(lpF, ltF, mpF, mtF, y)


def kernel(loc_preds, loc_targets, conf_preds, conf_targets,
           landmarks_preds, landmarks_targets):
    B, N = conf_targets.shape
    x0 = conf_preds[:, :, 0]
    x1 = conf_preds[:, :, 1]
    sc_out = _sc_conf(x0, x1, conf_targets)

    lpF = loc_preds.reshape(B, -1)
    ltF = loc_targets.reshape(B, -1)
    mpF = landmarks_preds.reshape(B, -1)
    mtF = landmarks_targets.reshape(B, -1)
    loc_p, lm_p = _tc_masked_sums(lpF, ltF, mpF, mtF, conf_targets)

    info = plsc.get_sparse_core_info()
    rpw = B // (info.num_cores * info.num_subcores)
    conf_sum = jnp.sum(sc_out[:, :rpw])
    num_matched = jnp.sum(sc_out[:, rpw:2 * rpw])
    return (jnp.sum(loc_p) + jnp.sum(lm_p) + conf_sum) / num_matched


# speedup vs baseline: 2.9423x; 2.9423x over previous
import functools

import jax
import jax.numpy as jnp
from jax import lax
from jax.experimental import pallas as pl
from jax.experimental.pallas import tpu as pltpu
from jax.experimental.pallas import tpu_sc as plsc

_LOG1P_C = (
    -2.31785471e-03, 1.53418978e-02, -4.76654862e-02, 9.54856631e-02,
    -1.45864737e-01, 1.93751659e-01, -2.48784242e-01, 3.33191908e-01,
    -4.99991423e-01, 9.99999795e-01, 8.18445333e-10,
)


def _log1p_poly(w):
    acc = jnp.full_like(w, _LOG1P_C[0])
    for c in _LOG1P_C[1:]:
        acc = acc * w + jnp.float32(c)
    return acc


def _sc_conf(x0, x1, y):
    B, N = y.shape
    info = plsc.get_sparse_core_info()
    nw = info.num_cores * info.num_subcores
    rpw = B // nw
    assert B % nw == 0 and N % 16 == 0
    nchunks = N // 16
    mesh = plsc.VectorSubcoreMesh(core_axis_name="c", subcore_axis_name="s")

    @functools.partial(
        pl.kernel,
        mesh=mesh,
        compiler_params=pltpu.CompilerParams(needs_layout_passes=False),
        out_type=jax.ShapeDtypeStruct((nw, 16), jnp.float32),
        scratch_types=[
            pltpu.VMEM((N,), jnp.float32),
            pltpu.VMEM((N,), jnp.float32),
            pltpu.VMEM((N,), jnp.int32),
            pltpu.VMEM((N,), jnp.float32),
            pltpu.VMEM((16,), jnp.float32),
        ],
    )
    def body(x0h, x1h, yh, outh, vx0, vx1, vy, vmine, vout):
        wid = lax.axis_index("s") * info.num_cores + lax.axis_index("c")
        lane = lax.broadcasted_iota(jnp.int32, (16,), 0)
        res = jnp.zeros((16,), jnp.float32)
        zf = jnp.zeros((16,), jnp.float32)
        for rl in range(rpw):
            r = wid * rpw + rl
            pltpu.sync_copy(x0h.at[r], vx0)
            pltpu.sync_copy(x1h.at[r], vx1)
            pltpu.sync_copy(yh.at[r], vy)

            def main_it(i, carry):
                npv, pcev, msv, zcv = carry
                sl = pl.ds(i * 16, 16)
                a0 = vx0[sl]
                a1 = vx1[sl]
                yy = vy[sl]
                t = a0 - a1
                pos = yy > 0
                ur = jnp.where(pos, jnp.maximum(t, 0.0), jnp.maximum(-t, 0.0))
                w = jnp.exp(-jnp.abs(t))
                ce = ur + _log1p_poly(w)
                mine = jnp.where(pos, jnp.float32(0.0), ce)
                vmine[sl] = mine
                one = jnp.float32(1.0)
                zero = jnp.float32(0.0)
                npv = npv + jnp.where(pos, one, zero)
                pcev = pcev + jnp.where(pos, ce, zero)
                msv = msv + mine
                zcv = zcv + jnp.where(mine == 0.0, one, zero)
                return npv, pcev, msv, zcv

            npv, pcev, msv, zcv = lax.fori_loop(
                0, nchunks, main_it, (zf, zf, zf, zf))
            np_s = jnp.sum(npv)
            pce_s = jnp.sum(pcev)
            ms_s = jnp.sum(msv)
            zc_s = jnp.sum(zcv)
            k = jnp.minimum(jnp.float32(3.0) * np_s, jnp.float32(N - 1))
            j = jnp.float32(N) - k

            def tail_zero():
                return jnp.float32(0.0)

            def tail_bisect():
                def bis(_, lohi):
                    lo, hi = lohi
                    mid = lo + (hi - lo) // 2

                    def cnt(i, acc):
                        kv = plsc.bitcast(vmine[pl.ds(i * 16, 16)], jnp.int32)
                        return acc + jnp.where(
                            kv < mid, jnp.float32(1.0), jnp.float32(0.0))

                    c = jnp.sum(lax.fori_loop(0, nchunks, cnt, zf))
                    take_hi = c < j
                    return (jnp.where(take_hi, mid, lo),
                            jnp.where(take_hi, hi, mid))

                lo, _ = lax.fori_loop(
                    0, 31, bis, (jnp.int32(0), jnp.int32(0x7F800001)))

                def fin(i, acc):
                    sb, cb = acc
                    mv = vmine[pl.ds(i * 16, 16)]
                    kv = plsc.bitcast(mv, jnp.int32)
                    ltm = kv < lo
                    sb = sb + jnp.where(ltm, mv, jnp.float32(0.0))
                    cb = cb + jnp.where(ltm, jnp.float32(1.0), jnp.float32(0.0))
                    return sb, cb

                sbv, cbv = lax.fori_loop(0, nchunks, fin, (zf, zf))
                vj = lax.bitcast_convert_type(lo, jnp.float32)
                return jnp.sum(sbv) + (j - jnp.sum(cbv)) * vj

            ss = lax.cond(j <= zc_s, tail_zero, tail_bisect)
            contrib = pce_s + (ms_s - ss)
            res = jnp.where(lane == rl, contrib, res)
            res = jnp.where(lane == rpw + rl, np_s, res)
        vout[...] = res
        pltpu.sync_copy(vout, outh.at[wid])

    return body(x0, x1, y)


def _sl1(x):
    ax = jnp.abs(x)
    return jnp.where(ax < 1.0, 0.5 * x * x, ax - 0.5)


def _tc_masked_sums(dlT, dmT, y):
    _, B, N = dlT.shape
    gb = B // 16
    grid = (gb,)

    def body(dl_ref, dm_ref, y_ref, ol_ref, om_ref):
        i = pl.program_id(0)
        posm = (y_ref[...] > 0).astype(jnp.float32)
        accl = jnp.zeros_like(posm)
        for d in range(4):
            accl = accl + _sl1(dl_ref[d].astype(jnp.float32))
        ol_ref[0, i] = jnp.sum(accl * posm)
        accm = jnp.zeros_like(posm)
        for d in range(10):
            accm = accm + _sl1(dm_ref[d].astype(jnp.float32))
        om_ref[0, i] = jnp.sum(accm * posm)

    return pl.pallas_call(
        body,
        grid=grid,
        in_specs=[
            pl.BlockSpec((4, 16, N), lambda i: (0, i, 0)),
            pl.BlockSpec((10, 16, N), lambda i: (0, i, 0)),
            pl.BlockSpec((16, N), lambda i: (i, 0)),
        ],
        out_specs=[
            pl.BlockSpec((1, gb), lambda i: (0, 0), memory_space=pltpu.SMEM),
            pl.BlockSpec((1, gb), lambda i: (0, 0), memory_space=pltpu.SMEM),
        ],
        out_shape=[
            jax.ShapeDtypeStruct((1, gb), jnp.float32),
            jax.ShapeDtypeStruct((1, gb), jnp.float32),
        ],
        compiler_params=pltpu.CompilerParams(
            dimension_semantics=("arbitrary",)),
    )(dlT, dmT, y)


def kernel(loc_preds, loc_targets, conf_preds, conf_targets,
           landmarks_preds, landmarks_targets):
    B, N = conf_targets.shape
    x0 = conf_preds[:, :, 0]
    x1 = conf_preds[:, :, 1]
    sc_out = _sc_conf(x0, x1, conf_targets)

    dlT = jnp.moveaxis(
        (loc_preds - loc_targets).astype(jnp.bfloat16), 2, 0)
    dmT = jnp.moveaxis(
        (landmarks_preds - landmarks_targets).astype(jnp.bfloat16), 2, 0)
    loc_p, lm_p = _tc_masked_sums(dlT, dmT, conf_targets)

    info = plsc.get_sparse_core_info()
    rpw = B // (info.num_cores * info.num_subcores)
    conf_sum = jnp.sum(sc_out[:, :rpw])
    num_matched = jnp.sum(sc_out[:, rpw:2 * rpw])
    return (jnp.sum(loc_p) + jnp.sum(lm_p) + conf_sum) / num_matched
